# baseline (speedup 1.0000x reference)
import functools

import jax
import jax.numpy as jnp
from jax import lax
from jax.experimental import pallas as pl
from jax.experimental.pallas import tpu as pltpu
from jax.experimental.pallas import tpu_sc as plsc

N = 10000
K = 32
D_IN = 128
D_OUT = 128

NUM_CORES = 2
NUM_SUBCORES = 16
NW = NUM_CORES * NUM_SUBCORES

CH = 80
NCHUNK = N // CH
MAXJ = -(-NCHUNK // NW)


def _sc_body(root_hbm, nbr_hbm, idx_hbm, out_hbm, acc_v, idx_v, sem):
    wid = lax.axis_index("s") * NUM_CORES + lax.axis_index("c")

    def do_chunk(i):
        start = i * CH
        pltpu.sync_copy(idx_hbm.at[i], idx_v)
        pltpu.sync_copy(root_hbm.at[pl.ds(start, CH)], acc_v)
        copies = [
            pltpu.async_copy(nbr_hbm.at[idx_v.at[k]], acc_v, sem, add=True)
            for k in range(K)
        ]
        for c in copies:
            c.wait()
        pltpu.sync_copy(acc_v, out_hbm.at[pl.ds(start, CH)])

    for j in range(MAXJ):
        i = wid + NW * j

        @pl.when(i < NCHUNK)
        def _():
            do_chunk(i)


def _sc_sum(root_feature, nbr_flat, idx):
    f = functools.partial(
        pl.kernel,
        out_type=jax.ShapeDtypeStruct((N, D_IN), jnp.float32),
        mesh=plsc.VectorSubcoreMesh(core_axis_name="c", subcore_axis_name="s"),
        scratch_types=[
            pltpu.VMEM((CH, D_IN), jnp.float32),
            pltpu.VMEM((K, CH), jnp.int32),
            pltpu.SemaphoreType.DMA,
        ],
    )(_sc_body)
    return f(root_feature, nbr_flat, idx)


def _mm_body(s_ref, w_ref, out_ref):
    w = w_ref[...] * (1.0 / (K + 1))
    out_ref[...] = jnp.maximum(
        jnp.dot(s_ref[...], w, preferred_element_type=jnp.float32), 0.0
    )


def _matmul_relu(sums, W):
    blk = 2000
    return pl.pallas_call(
        _mm_body,
        grid=(N // blk,),
        in_specs=[
            pl.BlockSpec((blk, D_IN), lambda i: (i, 0)),
            pl.BlockSpec((D_IN, D_OUT), lambda i: (0, 0)),
        ],
        out_specs=pl.BlockSpec((blk, D_OUT), lambda i: (i, 0)),
        out_shape=jax.ShapeDtypeStruct((N, D_OUT), jnp.float32),
    )(sums, W)


def kernel(root_feature, neighbor_features, W):
    nbr_flat = neighbor_features.reshape(N * K, D_IN)
    node_ids = jnp.arange(N, dtype=jnp.int32).reshape(NCHUNK, 1, CH)
    idx = node_ids * K + jnp.arange(K, dtype=jnp.int32)[None, :, None]
    sums = _sc_sum(root_feature, nbr_flat, idx)
    return _matmul_relu(sums, W)


# speedup vs baseline: 1.1422x; 1.1422x over previous
import functools

import jax
import jax.numpy as jnp
from jax import lax
from jax.experimental import pallas as pl
from jax.experimental.pallas import tpu as pltpu
from jax.experimental.pallas import tpu_sc as plsc

N = 10000
K = 32
D_IN = 128
D_OUT = 128

NT = 5600
NS = N - NT

NUM_CORES = 2
NUM_SUBCORES = 16
NW = NUM_CORES * NUM_SUBCORES

CH = 80
NCHUNK = NS // CH
MAXJ = -(-NCHUNK // NW)

_TC_BLK = 400


def _sc_body(root_hbm, nbr_hbm, idx_hbm, out_hbm, acc_v, idx_v, sem):
    wid = lax.axis_index("s") * NUM_CORES + lax.axis_index("c")

    def do_chunk(i):
        pltpu.sync_copy(idx_hbm.at[i], idx_v)
        pltpu.sync_copy(root_hbm.at[pl.ds(NT + i * CH, CH)], acc_v)
        copies = [
            pltpu.async_copy(nbr_hbm.at[idx_v.at[k]], acc_v, sem, add=True)
            for k in range(K)
        ]
        for c in copies:
            c.wait()
        pltpu.sync_copy(acc_v, out_hbm.at[pl.ds(i * CH, CH)])

    for j in range(MAXJ):
        i = wid + NW * j

        @pl.when(i < NCHUNK)
        def _():
            do_chunk(i)


def _sc_sum(root_feature, nbr_flat, idx):
    f = functools.partial(
        pl.kernel,
        out_type=jax.ShapeDtypeStruct((NS, D_IN), jnp.float32),
        mesh=plsc.VectorSubcoreMesh(core_axis_name="c", subcore_axis_name="s"),
        scratch_types=[
            pltpu.VMEM((CH, D_IN), jnp.float32),
            pltpu.VMEM((K, CH), jnp.int32),
            pltpu.SemaphoreType.DMA,
        ],
    )(_sc_body)
    return f(root_feature, nbr_flat, idx)


def _fused_body(root_ref, nbr_ref, w_ref, out_ref):
    s = jnp.sum(nbr_ref[...], axis=1) + root_ref[...]
    w = w_ref[...] * (1.0 / (K + 1))
    out_ref[...] = jnp.maximum(
        jnp.dot(s, w, preferred_element_type=jnp.float32), 0.0
    )


def _tc_fused(root_feature, neighbor_features, W):
    return pl.pallas_call(
        _fused_body,
        grid=(NT // _TC_BLK,),
        in_specs=[
            pl.BlockSpec((_TC_BLK, D_IN), lambda i: (i, 0)),
            pl.BlockSpec((_TC_BLK, K, D_IN), lambda i: (i, 0, 0)),
            pl.BlockSpec((D_IN, D_OUT), lambda i: (0, 0)),
        ],
        out_specs=pl.BlockSpec((_TC_BLK, D_OUT), lambda i: (i, 0)),
        out_shape=jax.ShapeDtypeStruct((NT, D_OUT), jnp.float32),
    )(root_feature, neighbor_features, W)


def _mm_body(s_ref, w_ref, out_ref):
    w = w_ref[...] * (1.0 / (K + 1))
    out_ref[...] = jnp.maximum(
        jnp.dot(s_ref[...], w, preferred_element_type=jnp.float32), 0.0
    )


def _matmul_relu(sums, W):
    return pl.pallas_call(
        _mm_body,
        grid=(1,),
        in_specs=[
            pl.BlockSpec((NS, D_IN), lambda i: (0, 0)),
            pl.BlockSpec((D_IN, D_OUT), lambda i: (0, 0)),
        ],
        out_specs=pl.BlockSpec((NS, D_OUT), lambda i: (0, 0)),
        out_shape=jax.ShapeDtypeStruct((NS, D_OUT), jnp.float32),
    )(sums, W)


def kernel(root_feature, neighbor_features, W):
    nbr_flat = neighbor_features.reshape(N * K, D_IN)
    node_ids = NT + jnp.arange(NS, dtype=jnp.int32).reshape(NCHUNK, 1, CH)
    idx = node_ids * K + jnp.arange(K, dtype=jnp.int32)[None, :, None]
    sums_s = _sc_sum(root_feature, nbr_flat, idx)
    out_t = _tc_fused(root_feature, neighbor_features, W)
    out_s = _matmul_relu(sums_s, W)
    return jnp.concatenate([out_t, out_s], axis=0)


# speedup vs baseline: 1.6810x; 1.4717x over previous
import jax
import jax.numpy as jnp
from jax.experimental import pallas as pl

N = 10000
K = 32
D_IN = 128
D_OUT = 128

_BLOCK = 1000


def _body(root_ref, nbr_ref, w_ref, out_ref):
    s = jnp.sum(nbr_ref[...], axis=1) + root_ref[...]
    w = w_ref[...] * (1.0 / (K + 1))
    out_ref[...] = jnp.maximum(
        jnp.dot(s, w, preferred_element_type=jnp.float32), 0.0
    )


def kernel(root_feature, neighbor_features, W):
    return pl.pallas_call(
        _body,
        grid=(N // _BLOCK,),
        in_specs=[
            pl.BlockSpec((_BLOCK, D_IN), lambda i: (i, 0)),
            pl.BlockSpec((_BLOCK, K, D_IN), lambda i: (i, 0, 0)),
            pl.BlockSpec((D_IN, D_OUT), lambda i: (0, 0)),
        ],
        out_specs=pl.BlockSpec((_BLOCK, D_OUT), lambda i: (i, 0)),
        out_shape=jax.ShapeDtypeStruct((N, D_OUT), jnp.float32),
    )(root_feature, neighbor_features, W)
